# baseline (speedup 1.0000x reference)
import functools

import jax
import jax.numpy as jnp
from jax import lax
from jax.experimental import pallas as pl
from jax.experimental.pallas import tpu as pltpu
from jax.experimental.pallas import tpu_sc as plsc

NC = 2
NT = 16
LANES = 16
CHUNK = 128
IDX_BLK = 24
DH = 64
RING = 4


def _build_sc_call(n, n_pad, n_chunks):
  acc_rows = 2 * n_pad
  cpt = n_chunks // NT
  nblk = cpt // IDX_BLK
  assert cpt % IDX_BLK == 0 and IDX_BLK % RING == 0 and nblk >= 2
  assert n % NT == 0
  rows_per_tile = acc_rows // NT
  out_rows_per_tile = n // NT

  mesh = plsc.VectorSubcoreMesh(core_axis_name="c", subcore_axis_name="s")

  @functools.partial(
      pl.kernel,
      mesh=mesh,
      compiler_params=pltpu.CompilerParams(use_tc_tiling_on_sc=False),
      out_type=jax.ShapeDtypeStruct((n, 2 * NC * DH), jnp.float32),
      scratch_types=[
          pltpu.VMEM((2, IDX_BLK, 2, CHUNK), jnp.int32),
          pltpu.VMEM((RING, CHUNK), jnp.int32),
          pltpu.VMEM((RING * CHUNK, DH), jnp.float32),
          pltpu.VMEM_SHARED((acc_rows, DH), jnp.float32),
          pltpu.SemaphoreType.DMA((2,)),
          pltpu.SemaphoreType.DMA((2,)),
          pltpu.SemaphoreType.DMA((2,)),
      ],
  )
  def sc_kernel(x_hbm, arr_hbm, out_hbm,
                idxblk, srcb, rows, acc, isem, gsem, ssem):
    cid = lax.axis_index("c")
    tid = lax.axis_index("s")

    def i_desc(bi):
      h = bi & 1
      return pltpu.make_async_copy(
          arr_hbm.at[pl.ds(tid * cpt + bi * IDX_BLK, IDX_BLK)], idxblk.at[h],
          isem.at[h])

    i_desc(0).start()

    def zrow(r, carry):
      for j in range(DH // LANES):
        rows[r, pl.ds(j * LANES, LANES)] = jnp.zeros((LANES,), jnp.float32)
      return carry

    lax.fori_loop(0, CHUNK, zrow, 0)

    def zcp(q, carry):
      pltpu.sync_copy(rows.at[pl.ds(0, CHUNK)],
                      acc.at[pl.ds(tid * rows_per_tile + q * CHUNK, CHUNK)])
      return carry

    lax.fori_loop(0, rows_per_tile // CHUNK, zcp, 0)
    i_desc(0).wait()
    plsc.subcore_barrier()

    row_base = cid * n

    def build_src(k, b):
      h = (k // IDX_BLK) & 1
      kl = k % IDX_BLK
      for j in range(CHUNK // LANES):
        v = idxblk[h, kl, 0, pl.ds(j * LANES, LANES)]
        srcb[b, pl.ds(j * LANES, LANES)] = v + row_base

    def g_start(k, b, q):
      pltpu.make_async_copy(x_hbm.at[srcb.at[b]],
                            rows.at[pl.ds(b * CHUNK, CHUNK)],
                            gsem.at[q]).start()

    def g_wait_pair(q):
      pltpu.make_async_copy(x_hbm.at[pl.ds(0, 2 * CHUNK)],
                            rows.at[pl.ds(q * 2 * CHUNK, 2 * CHUNK)],
                            gsem.at[q]).wait()

    def s_start(k, b, q):
      h = (k // IDX_BLK) & 1
      kl = k % IDX_BLK
      pltpu.make_async_copy(rows.at[pl.ds(b * CHUNK, CHUNK)],
                            acc.at[idxblk.at[h, kl, 1]],
                            ssem.at[q]).start(add=True)

    def s_wait_pair(q):
      pltpu.make_async_copy(x_hbm.at[pl.ds(0, 2 * CHUNK)],
                            acc.at[pl.ds(0, 2 * CHUNK)],
                            ssem.at[q]).wait()

    for b in range(2):
      build_src(b, b)
      g_start(b, b, 0)

    npairs = cpt // 2
    blk_pairs = IDX_BLK // 2

    def pair_body(p, carry):
      q = p & 1
      k0 = 2 * p
      g_wait_pair(q)

      @pl.when(p >= 1)
      def _():
        s_wait_pair(1 - q)

      s_start(k0, 2 * q, q)
      s_start(k0 + 1, 2 * q + 1, q)

      @pl.when(p + 1 < npairs)
      def _():
        qn = 1 - q
        kn = k0 + 2

        @pl.when(p % blk_pairs == blk_pairs - 1)
        def _():
          i_desc(kn // IDX_BLK).wait()

        build_src(kn, 2 * qn)
        build_src(kn + 1, 2 * qn + 1)
        g_start(kn, 2 * qn, qn)
        g_start(kn + 1, 2 * qn + 1, qn)

      @pl.when(p % blk_pairs == 2)
      def _():
        bi2 = p // blk_pairs + 1

        @pl.when(bi2 < nblk)
        def _():
          i_desc(bi2).start()

      return carry

    lax.fori_loop(0, npairs, pair_body, 0)
    s_wait_pair((npairs - 1) & 1)

    plsc.subcore_barrier()
    r0 = tid * out_rows_per_tile
    pltpu.sync_copy(acc.at[pl.ds(r0, out_rows_per_tile)],
                    out_hbm.at[pl.ds(r0, out_rows_per_tile),
                               pl.ds(cid * DH, DH)])
    pltpu.sync_copy(acc.at[pl.ds(n_pad + r0, out_rows_per_tile)],
                    out_hbm.at[pl.ds(r0, out_rows_per_tile),
                               pl.ds(2 * DH + cid * DH, DH)])

  return sc_kernel


def kernel(x, edge_index, edge_index2):
  n, d = x.shape
  assert d == 2 * DH
  n_pad = ((n + 1023) // 1024) * 1024
  dummy = n_pad - 1

  x2h = jnp.concatenate([x[:, :DH], x[:, DH:]], axis=0)

  src = jnp.concatenate([edge_index[1], edge_index2[1]])
  dst = jnp.concatenate([edge_index[0], edge_index2[0] + n_pad])
  e_tot = src.shape[0]
  grain = NT * IDX_BLK * CHUNK
  e_pad = ((e_tot + grain - 1) // grain) * grain
  src = jnp.pad(src, (0, e_pad - e_tot))
  dst = jnp.pad(dst, (0, e_pad - e_tot), constant_values=dummy)
  n_chunks = e_pad // CHUNK
  arr = jnp.stack([src.reshape(n_chunks, CHUNK), dst.reshape(n_chunks, CHUNK)],
                  axis=1)

  sc_call = _build_sc_call(n, n_pad, n_chunks)
  return sc_call(x2h, arr)


# speedup vs baseline: 1.1918x; 1.1918x over previous
import functools

import jax
import jax.numpy as jnp
from jax import lax
from jax.experimental import pallas as pl
from jax.experimental.pallas import tpu as pltpu
from jax.experimental.pallas import tpu_sc as plsc

NC = 2
NT = 16
LANES = 16
CHUNK = 128
IDX_BLK = 24
DH = 64
RING = 4


def _build_sc_call(n, n_pad, n_chunks):
  acc_rows = 2 * n_pad
  cpt = n_chunks // NT
  nblk = cpt // IDX_BLK
  assert cpt % IDX_BLK == 0 and IDX_BLK % RING == 0 and nblk >= 2
  assert n % NT == 0
  rows_per_tile = acc_rows // NT
  out_rows_per_tile = n // NT

  mesh = plsc.VectorSubcoreMesh(core_axis_name="c", subcore_axis_name="s")

  @functools.partial(
      pl.kernel,
      mesh=mesh,
      compiler_params=pltpu.CompilerParams(use_tc_tiling_on_sc=False),
      out_type=jax.ShapeDtypeStruct((n, 2 * NC * DH), jnp.float32),
      scratch_types=[
          pltpu.VMEM((2, IDX_BLK, 2, CHUNK), jnp.int32),
          pltpu.VMEM((RING, CHUNK), jnp.int32),
          pltpu.VMEM((RING, CHUNK, DH), jnp.float32),
          pltpu.VMEM_SHARED((acc_rows, DH), jnp.float32),
          pltpu.SemaphoreType.DMA((2,)),
          pltpu.SemaphoreType.DMA((RING,)),
          pltpu.SemaphoreType.DMA((RING,)),
      ],
  )
  def sc_kernel(x_hbm, arr_hbm, out_hbm,
                idxblk, srcb, rows, acc, isem, gsem, ssem):
    cid = lax.axis_index("c")
    tid = lax.axis_index("s")

    def i_desc(bi):
      h = bi & 1
      return pltpu.make_async_copy(
          arr_hbm.at[pl.ds(tid * cpt + bi * IDX_BLK, IDX_BLK)], idxblk.at[h],
          isem.at[h])

    i_desc(0).start()

    zbuf = rows.at[0]

    def zrow(r, carry):
      for j in range(DH // LANES):
        zbuf[r, pl.ds(j * LANES, LANES)] = jnp.zeros((LANES,), jnp.float32)
      return carry

    lax.fori_loop(0, CHUNK, zrow, 0)

    def zcp(q, carry):
      pltpu.sync_copy(zbuf, acc.at[pl.ds(tid * rows_per_tile + q * CHUNK, CHUNK)])
      return carry

    lax.fori_loop(0, rows_per_tile // CHUNK, zcp, 0)
    i_desc(0).wait()
    plsc.subcore_barrier()

    row_base = cid * n

    def build_src(k, b):
      h = (k // IDX_BLK) & 1
      kl = k % IDX_BLK
      for j in range(CHUNK // LANES):
        v = idxblk[h, kl, 0, pl.ds(j * LANES, LANES)]
        srcb[b, pl.ds(j * LANES, LANES)] = v + row_base

    def g_desc(b):
      return pltpu.make_async_copy(x_hbm.at[srcb.at[b]], rows.at[b], gsem.at[b])

    def s_desc(k, b):
      h = (k // IDX_BLK) & 1
      kl = k % IDX_BLK
      return pltpu.make_async_copy(rows.at[b], acc.at[idxblk.at[h, kl, 1]],
                                   ssem.at[b])

    for b in range(2):
      build_src(b, b)
      g_desc(b).start()

    def chunk_body(k, carry):
      kp = k + 2

      @pl.when(kp < cpt)
      def _():
        bn = kp & 3

        @pl.when(k >= 2)
        def _():
          s_desc(k - 2, bn).wait()

        @pl.when(kp % IDX_BLK == 0)
        def _():
          i_desc(kp // IDX_BLK).wait()

        build_src(kp, bn)
        g_desc(bn).start()

      @pl.when(k % IDX_BLK == 4)
      def _():
        bi2 = k // IDX_BLK + 1

        @pl.when(bi2 < nblk)
        def _():
          i_desc(bi2).start()

      b = k & 3
      g_desc(b).wait()
      s_desc(k, b).start(add=True)
      return carry

    lax.fori_loop(0, cpt, chunk_body, 0)
    for k in range(cpt - RING, cpt):
      s_desc(k, k & 3).wait()

    plsc.subcore_barrier()
    r0 = tid * out_rows_per_tile
    pltpu.sync_copy(acc.at[pl.ds(r0, out_rows_per_tile)],
                    out_hbm.at[pl.ds(r0, out_rows_per_tile),
                               pl.ds(cid * DH, DH)])
    pltpu.sync_copy(acc.at[pl.ds(n_pad + r0, out_rows_per_tile)],
                    out_hbm.at[pl.ds(r0, out_rows_per_tile),
                               pl.ds(2 * DH + cid * DH, DH)])

  return sc_kernel


def kernel(x, edge_index, edge_index2):
  n, d = x.shape
  assert d == 2 * DH
  n_pad = ((n + 1023) // 1024) * 1024
  dummy = n_pad - 1

  x2h = jnp.concatenate([x[:, :DH], x[:, DH:]], axis=0)

  src = jnp.concatenate([edge_index[1], edge_index2[1]])
  dst = jnp.concatenate([edge_index[0], edge_index2[0] + n_pad])
  e_tot = src.shape[0]
  grain = NT * IDX_BLK * CHUNK
  e_pad = ((e_tot + grain - 1) // grain) * grain
  src = jnp.pad(src, (0, e_pad - e_tot))
  dst = jnp.pad(dst, (0, e_pad - e_tot), constant_values=dummy)
  n_chunks = e_pad // CHUNK
  arr = jnp.stack([src.reshape(n_chunks, CHUNK), dst.reshape(n_chunks, CHUNK)],
                  axis=1)

  sc_call = _build_sc_call(n, n_pad, n_chunks)
  return sc_call(x2h, arr)
